# baseline (speedup 1.0000x reference)
import functools

import jax
import jax.numpy as jnp
from jax import lax
from jax.experimental import pallas as pl
from jax.experimental.pallas import tpu as pltpu
from jax.experimental.pallas import tpu_sc as plsc

P = 1000
E = 512
F = 20
SEQ = 2048
TOT = P + SEQ

_NC = 2
_NS = 16
_NW = _NC * _NS

_GATHER_WORKERS = 25
_GATHER_ROWS = P // _GATHER_WORKERS
_EMB_ROWS = SEQ // _NW


def _y_body(table_hbm, labels_hbm, emby_hbm, out_hbm, idx_v, rows_v, buf_v,
            gsem):
    wid = lax.axis_index("s") * _NC + lax.axis_index("c")

    @pl.when(wid < _GATHER_WORKERS)
    def _():
        base = wid * _GATHER_ROWS
        pltpu.sync_copy(labels_hbm.at[pl.ds(base, _GATHER_ROWS)], idx_v)
        pltpu.async_copy(table_hbm.at[idx_v], rows_v, gsem).wait()
        pltpu.sync_copy(rows_v, out_hbm.at[pl.ds(base, _GATHER_ROWS)])

    ebase = wid * _EMB_ROWS
    pltpu.sync_copy(emby_hbm.at[pl.ds(ebase, _EMB_ROWS)], buf_v)
    pltpu.sync_copy(buf_v, out_hbm.at[pl.ds(P + ebase, _EMB_ROWS)])


@functools.cache
def _y_concat():
    return pl.kernel(
        _y_body,
        out_type=jax.ShapeDtypeStruct((TOT, E), jnp.float32),
        mesh=plsc.VectorSubcoreMesh(core_axis_name="c", subcore_axis_name="s"),
        scratch_types=[
            pltpu.VMEM((_GATHER_ROWS,), jnp.int32),
            pltpu.VMEM((_GATHER_ROWS, E), jnp.float32),
            pltpu.VMEM((_EMB_ROWS, E), jnp.float32),
            pltpu.SemaphoreType.DMA,
        ],
    )

_XB = 1000
_NXB = 4


def _x_body(tune_ref, emb_ref, out_ref):
    i = pl.program_id(1)

    @pl.when(i == 0)
    def _():
        out_ref[...] = tune_ref[...]

    @pl.when(i > 0)
    def _():
        out_ref[...] = emb_ref[...]


_x_concat = pl.pallas_call(
    _x_body,
    grid=(F, _NXB),
    in_specs=[
        pl.BlockSpec((1, 1, _XB, E), lambda f, i: (0, f, 0, 0)),
        pl.BlockSpec((1, 1, _XB, E),
                     lambda f, i: (0, f, jnp.maximum(i - 1, 0), 0)),
    ],
    out_specs=pl.BlockSpec((1, 1, _XB, E), lambda f, i: (0, f, i, 0)),
    out_shape=jax.ShapeDtypeStruct((1, F, TOT, E), jnp.float32),
)


def kernel(embedding_X, embedding_y, tune_X, tune_y_table, labels):
    modifiedy = _y_concat()(
        tune_y_table,
        labels.reshape(P).astype(jnp.int32),
        embedding_y.reshape(SEQ, E),
    ).reshape(1, TOT, E)
    modifiedX = jnp.transpose(
        _x_concat(jnp.transpose(tune_X, (0, 2, 1, 3)),
                  jnp.transpose(embedding_X, (0, 2, 1, 3))),
        (0, 2, 1, 3))
    return (modifiedX, modifiedy)


# speedup vs baseline: 1.1766x; 1.1766x over previous
import functools

import jax
import jax.numpy as jnp
from jax import lax
from jax.experimental import pallas as pl
from jax.experimental.pallas import tpu as pltpu
from jax.experimental.pallas import tpu_sc as plsc

P = 1000
E = 512
F = 20
SEQ = 2048
TOT = P + SEQ

_NC = 2
_NS = 16
_NW = _NC * _NS

_GATHER_WORKERS = 25
_GATHER_ROWS = P // _GATHER_WORKERS
_EMB_ROWS = SEQ // _NW


def _y_body(table_hbm, labels_hbm, emby_hbm, out_hbm, idx_v, rows_v, buf_v,
            gsem):
    wid = lax.axis_index("s") * _NC + lax.axis_index("c")

    @pl.when(wid < _GATHER_WORKERS)
    def _():
        base = wid * _GATHER_ROWS
        pltpu.sync_copy(labels_hbm.at[pl.ds(base, _GATHER_ROWS)], idx_v)
        pltpu.async_copy(table_hbm.at[idx_v], rows_v, gsem).wait()
        pltpu.sync_copy(rows_v, out_hbm.at[pl.ds(base, _GATHER_ROWS)])

    ebase = wid * _EMB_ROWS
    pltpu.sync_copy(emby_hbm.at[pl.ds(ebase, _EMB_ROWS)], buf_v)
    pltpu.sync_copy(buf_v, out_hbm.at[pl.ds(P + ebase, _EMB_ROWS)])


@functools.cache
def _y_concat():
    return pl.kernel(
        _y_body,
        out_type=jax.ShapeDtypeStruct((TOT, E), jnp.float32),
        mesh=plsc.VectorSubcoreMesh(core_axis_name="c", subcore_axis_name="s"),
        scratch_types=[
            pltpu.VMEM((_GATHER_ROWS,), jnp.int32),
            pltpu.VMEM((_GATHER_ROWS, E), jnp.float32),
            pltpu.VMEM((_EMB_ROWS, E), jnp.float32),
            pltpu.SemaphoreType.DMA,
        ],
    )

_XB = 1000
_NXB = 4


def _x_body(tune_ref, emb_ref, out_ref):
    i = pl.program_id(1)

    @pl.when(i == 0)
    def _():
        out_ref[...] = tune_ref[...]

    @pl.when(i > 0)
    def _():
        out_ref[...] = emb_ref[...]


_FB = 2
_x_concat = pl.pallas_call(
    _x_body,
    grid=(F // _FB, _NXB),
    in_specs=[
        pl.BlockSpec((1, _FB, _XB, E), lambda f, i: (0, f, 0, 0)),
        pl.BlockSpec((1, _FB, _XB, E),
                     lambda f, i: (0, f, jnp.maximum(i - 1, 0), 0)),
    ],
    out_specs=pl.BlockSpec((1, _FB, _XB, E), lambda f, i: (0, f, i, 0)),
    out_shape=jax.ShapeDtypeStruct((1, F, TOT, E), jnp.float32),
)


def kernel(embedding_X, embedding_y, tune_X, tune_y_table, labels):
    modifiedy = _y_concat()(
        tune_y_table,
        labels.reshape(P).astype(jnp.int32),
        embedding_y.reshape(SEQ, E),
    ).reshape(1, TOT, E)
    modifiedX = jnp.transpose(
        _x_concat(jnp.transpose(tune_X, (0, 2, 1, 3)),
                  jnp.transpose(embedding_X, (0, 2, 1, 3))),
        (0, 2, 1, 3))
    return (modifiedX, modifiedy)


# speedup vs baseline: 1.3291x; 1.1295x over previous
import functools

import jax
import jax.numpy as jnp
from jax import lax
from jax.experimental import pallas as pl
from jax.experimental.pallas import tpu as pltpu
from jax.experimental.pallas import tpu_sc as plsc

P = 1000
E = 512
F = 20
SEQ = 2048
TOT = P + SEQ

_NC = 2
_NS = 16
_NW = _NC * _NS

_GATHER_WORKERS = 25
_GATHER_ROWS = P // _GATHER_WORKERS
_EMB_ROWS = SEQ // _NW


def _y_body(table_hbm, labels_hbm, emby_hbm, out_hbm, idx_v, rows_v, buf_v,
            gsem):
    wid = lax.axis_index("s") * _NC + lax.axis_index("c")

    @pl.when(wid < _GATHER_WORKERS)
    def _():
        base = wid * _GATHER_ROWS
        pltpu.sync_copy(labels_hbm.at[pl.ds(base, _GATHER_ROWS)], idx_v)
        pltpu.async_copy(table_hbm.at[idx_v], rows_v, gsem).wait()
        pltpu.sync_copy(rows_v, out_hbm.at[pl.ds(base, _GATHER_ROWS)])

    ebase = wid * _EMB_ROWS
    pltpu.sync_copy(emby_hbm.at[pl.ds(ebase, _EMB_ROWS)], buf_v)
    pltpu.sync_copy(buf_v, out_hbm.at[pl.ds(P + ebase, _EMB_ROWS)])


@functools.cache
def _y_concat():
    return pl.kernel(
        _y_body,
        out_type=jax.ShapeDtypeStruct((TOT, E), jnp.float32),
        mesh=plsc.VectorSubcoreMesh(core_axis_name="c", subcore_axis_name="s"),
        scratch_types=[
            pltpu.VMEM((_GATHER_ROWS,), jnp.int32),
            pltpu.VMEM((_GATHER_ROWS, E), jnp.float32),
            pltpu.VMEM((_EMB_ROWS, E), jnp.float32),
            pltpu.SemaphoreType.DMA,
        ],
    )

_XB = 1000
_NXB = 4


def _x_body(tune_ref, emb_ref, out_ref):
    i = pl.program_id(1)

    @pl.when(i == 0)
    def _():
        out_ref[...] = tune_ref[...]

    @pl.when(i > 0)
    def _():
        out_ref[...] = emb_ref[...]


_FB = 4
_x_concat = pl.pallas_call(
    _x_body,
    grid=(F // _FB, _NXB),
    in_specs=[
        pl.BlockSpec((1, _FB, _XB, E), lambda f, i: (0, f, 0, 0)),
        pl.BlockSpec((1, _FB, _XB, E),
                     lambda f, i: (0, f, jnp.maximum(i - 1, 0), 0)),
    ],
    out_specs=pl.BlockSpec((1, _FB, _XB, E), lambda f, i: (0, f, i, 0)),
    out_shape=jax.ShapeDtypeStruct((1, F, TOT, E), jnp.float32),
)


def kernel(embedding_X, embedding_y, tune_X, tune_y_table, labels):
    modifiedy = _y_concat()(
        tune_y_table,
        labels.reshape(P).astype(jnp.int32),
        embedding_y.reshape(SEQ, E),
    ).reshape(1, TOT, E)
    modifiedX = jnp.transpose(
        _x_concat(jnp.transpose(tune_X, (0, 2, 1, 3)),
                  jnp.transpose(embedding_X, (0, 2, 1, 3))),
        (0, 2, 1, 3))
    return (modifiedX, modifiedy)


# speedup vs baseline: 1.4396x; 1.0831x over previous
import functools

import jax
import jax.numpy as jnp
from jax import lax
from jax.experimental import pallas as pl
from jax.experimental.pallas import tpu as pltpu
from jax.experimental.pallas import tpu_sc as plsc

P = 1000
E = 512
F = 20
SEQ = 2048
TOT = P + SEQ

_NC = 2
_NS = 16
_NW = _NC * _NS

_GATHER_WORKERS = 25
_GATHER_ROWS = P // _GATHER_WORKERS
_EMB_ROWS = SEQ // _NW


def _y_body(table_hbm, labels_hbm, emby_hbm, out_hbm, idx_v, rows_v, buf_v,
            gsem):
    wid = lax.axis_index("s") * _NC + lax.axis_index("c")

    @pl.when(wid < _GATHER_WORKERS)
    def _():
        base = wid * _GATHER_ROWS
        pltpu.sync_copy(labels_hbm.at[pl.ds(base, _GATHER_ROWS)], idx_v)
        pltpu.async_copy(table_hbm.at[idx_v], rows_v, gsem).wait()
        pltpu.sync_copy(rows_v, out_hbm.at[pl.ds(base, _GATHER_ROWS)])

    ebase = wid * _EMB_ROWS
    pltpu.sync_copy(emby_hbm.at[pl.ds(ebase, _EMB_ROWS)], buf_v)
    pltpu.sync_copy(buf_v, out_hbm.at[pl.ds(P + ebase, _EMB_ROWS)])


@functools.cache
def _y_concat():
    return pl.kernel(
        _y_body,
        out_type=jax.ShapeDtypeStruct((TOT, E), jnp.float32),
        mesh=plsc.VectorSubcoreMesh(core_axis_name="c", subcore_axis_name="s"),
        scratch_types=[
            pltpu.VMEM((_GATHER_ROWS,), jnp.int32),
            pltpu.VMEM((_GATHER_ROWS, E), jnp.float32),
            pltpu.VMEM((_EMB_ROWS, E), jnp.float32),
            pltpu.SemaphoreType.DMA,
        ],
    )

_HR = SEQ // 2
_NTC = F
_NCH = F + 2 * F
_NBUF = 16
_DEPTH = 8


def _x_chunk(k, tune_ref, emb_ref, out_ref, buf, in_sems, out_sems,
             action):
    b = lax.rem(k, _NBUF)

    @pl.when(k < _NTC)
    def _():
        src = tune_ref.at[0, k]
        dst = out_ref.at[0, k, pl.ds(0, P)]
        if action == "in_start":
            pltpu.make_async_copy(src, buf.at[b, pl.ds(0, P)],
                                  in_sems.at[b]).start()
        elif action == "in_wait":
            pltpu.make_async_copy(src, buf.at[b, pl.ds(0, P)],
                                  in_sems.at[b]).wait()
        elif action == "out_start":
            pltpu.make_async_copy(buf.at[b, pl.ds(0, P)], dst,
                                  out_sems.at[b]).start()
        else:
            pltpu.make_async_copy(buf.at[b, pl.ds(0, P)], dst,
                                  out_sems.at[b]).wait()

    @pl.when(k >= _NTC)
    def _():
        e = k - _NTC
        f = lax.div(e, 2)
        h = lax.rem(e, 2)
        src = emb_ref.at[0, f, pl.ds(h * _HR, _HR)]
        dst = out_ref.at[0, f, pl.ds(P + h * _HR, _HR)]
        if action == "in_start":
            pltpu.make_async_copy(src, buf.at[b], in_sems.at[b]).start()
        elif action == "in_wait":
            pltpu.make_async_copy(src, buf.at[b], in_sems.at[b]).wait()
        elif action == "out_start":
            pltpu.make_async_copy(buf.at[b], dst, out_sems.at[b]).start()
        else:
            pltpu.make_async_copy(buf.at[b], dst, out_sems.at[b]).wait()


def _x_body(tune_ref, emb_ref, out_ref, buf, in_sems, out_sems):
    args = (tune_ref, emb_ref, out_ref, buf, in_sems, out_sems)

    for k in range(_DEPTH):
        _x_chunk(k, *args, "in_start")

    def loop(k, carry):
        _x_chunk(k, *args, "in_wait")
        _x_chunk(k, *args, "out_start")
        j = k + _DEPTH

        @pl.when(j < _NCH)
        def _():
            m = j - _NBUF

            @pl.when(m >= 0)
            def _():
                _x_chunk(m, *args, "out_wait")

            _x_chunk(j, *args, "in_start")

        return carry

    lax.fori_loop(0, _NCH, loop, 0)

    for m in range(_NCH - _NBUF, _NCH):
        _x_chunk(m, *args, "out_wait")


_x_concat = pl.pallas_call(
    _x_body,
    in_specs=[pl.BlockSpec(memory_space=pl.ANY),
              pl.BlockSpec(memory_space=pl.ANY)],
    out_specs=pl.BlockSpec(memory_space=pl.ANY),
    out_shape=jax.ShapeDtypeStruct((1, F, TOT, E), jnp.float32),
    scratch_shapes=[
        pltpu.VMEM((_NBUF, _HR, E), jnp.float32),
        pltpu.SemaphoreType.DMA((_NBUF,)),
        pltpu.SemaphoreType.DMA((_NBUF,)),
    ],
)


def kernel(embedding_X, embedding_y, tune_X, tune_y_table, labels):
    modifiedy = _y_concat()(
        tune_y_table,
        labels.reshape(P).astype(jnp.int32),
        embedding_y.reshape(SEQ, E),
    ).reshape(1, TOT, E)
    modifiedX = jnp.transpose(
        _x_concat(jnp.transpose(tune_X, (0, 2, 1, 3)),
                  jnp.transpose(embedding_X, (0, 2, 1, 3))),
        (0, 2, 1, 3))
    return (modifiedX, modifiedy)
